# Initial kernel scaffold; baseline (speedup 1.0000x reference)
import jax
import jax.numpy as jnp
from jax.experimental import pallas as pl


def kernel(x, W_embed, b_embed, pe, W0, b0, W1, b1, W_out, b_out, edge_index):
    raise NotImplementedError("write your pallas kernel here")


# speedup vs baseline: 3.3415x; 3.3415x over previous
import jax
import jax.numpy as jnp
from jax import lax
from jax.experimental import pallas as pl
from jax.experimental.pallas import tpu as pltpu
from jax.experimental.pallas import tpu_sc as plsc

C = 3
IMG = 224
GRID = 14
P = 16
NP = 196
PD = C * P * P
HID = 96
ENC = 96
E0 = 2 * 2 * GRID * (GRID - 1)
BB = 8
NW = 32
GG = NP * HID


def _dinv_body(ei_ref, dv_ref):
    dst = ei_ref[1:2, :]
    rows = jax.lax.broadcasted_iota(jnp.int32, (NP, E0), 0)
    oh_d = (rows == dst).astype(jnp.float32)
    deg = 1.0 + jnp.sum(oh_d, axis=1, keepdims=True)
    dinv = jax.lax.rsqrt(deg)
    dv_ref[...] = jnp.concatenate([dinv] * BB, axis=0)


def _tc1_body(xp_ref, we_ref, pet_ref, w0_ref, dv_ref, z_ref):
    h = jnp.dot(xp_ref[...], we_ref[...],
                preferred_element_type=jnp.float32) + pet_ref[...]
    z_ref[...] = jnp.dot(h, w0_ref[...],
                         preferred_element_type=jnp.float32) * dv_ref[...]


def _tc2_body(y_ref, dv_ref, b0_ref, w1_ref, z_ref):
    h = jax.nn.relu(y_ref[...] * dv_ref[...] + b0_ref[...])
    z_ref[...] = jnp.dot(h, w1_ref[...],
                         preferred_element_type=jnp.float32) * dv_ref[...]


def _tc3_body(y_ref, dv_ref, b1_ref, wo_ref, bo_ref, o_ref):
    h = jax.nn.relu(y_ref[...] * dv_ref[...] + b1_ref[...])
    o_ref[...] = jnp.dot(h, wo_ref[...],
                         preferred_element_type=jnp.float32) + bo_ref[...]


def _sc_agg_body(z_hbm, y_hbm, z_v, y_v, sem_in, sem_out):
    wid = lax.axis_index("s") * 2 + lax.axis_index("c")
    gpw = 128 // NW

    def per_graph(g, _):
        base = (wid * gpw + g) * GG
        pltpu.async_copy(z_hbm.at[pl.ds(base, GG)], z_v, sem_in).wait()

        def row_i(i, _):
            m_u = (i > 0).astype(jnp.float32)
            m_d = (i < GRID - 1).astype(jnp.float32)

            def node_j(j, _):
                m_l = (j > 0).astype(jnp.float32)
                m_r = (j < GRID - 1).astype(jnp.float32)
                n = i * GRID + j
                t = n * HID
                t_u = jnp.maximum(n - GRID, 0) * HID
                t_d = jnp.minimum(n + GRID, NP - 1) * HID
                t_l = jnp.maximum(n - 1, 0) * HID
                t_r = jnp.minimum(n + 1, NP - 1) * HID
                for c in range(HID // 16):
                    o = c * 16
                    v = z_v[pl.ds(t + o, 16)]
                    v = v + m_u * z_v[pl.ds(t_u + o, 16)]
                    v = v + m_d * z_v[pl.ds(t_d + o, 16)]
                    v = v + m_l * z_v[pl.ds(t_l + o, 16)]
                    v = v + m_r * z_v[pl.ds(t_r + o, 16)]
                    y_v[pl.ds(t + o, 16)] = v
                return 0

            lax.fori_loop(0, GRID, node_j, 0)
            return 0

        lax.fori_loop(0, GRID, row_i, 0)
        pltpu.async_copy(y_v, y_hbm.at[pl.ds(base, GG)], sem_out).wait()
        return 0

    lax.fori_loop(0, 128 // NW, per_graph, 0)


def _sc_agg(z_flat):
    mesh = plsc.VectorSubcoreMesh(core_axis_name="c", subcore_axis_name="s")
    fn = pl.kernel(
        _sc_agg_body,
        out_type=jax.ShapeDtypeStruct((128 * GG,), jnp.float32),
        mesh=mesh,
        scratch_types=[
            pltpu.VMEM((GG,), jnp.float32),
            pltpu.VMEM((GG,), jnp.float32),
            pltpu.SemaphoreType.DMA,
            pltpu.SemaphoreType.DMA,
        ],
    )
    return fn(z_flat)


def kernel(x, W_embed, b_embed, pe, W0, b0, W1, b1, W_out, b_out, edge_index):
    bsz = x.shape[0]
    xp = x.reshape(bsz, C, GRID, P, GRID, P)
    xp = xp.transpose(0, 2, 4, 1, 3, 5).reshape(bsz * NP, PD)
    pet = jnp.tile(pe + b_embed[None, :], (BB, 1))
    ei0 = edge_index[:, :E0]
    grid = (bsz // BB,)
    full = lambda i: (0, 0)
    blk = lambda i: (i, 0)

    dinv_col = pl.pallas_call(
        _dinv_body,
        in_specs=[pl.BlockSpec((2, E0), lambda: (0, 0))],
        out_specs=pl.BlockSpec((BB * NP, 1), lambda: (0, 0)),
        out_shape=jax.ShapeDtypeStruct((BB * NP, 1), jnp.float32),
    )(ei0)

    z0 = pl.pallas_call(
        _tc1_body,
        grid=grid,
        in_specs=[
            pl.BlockSpec((BB * NP, PD), blk),
            pl.BlockSpec((PD, HID), full),
            pl.BlockSpec((BB * NP, HID), full),
            pl.BlockSpec((HID, HID), full),
            pl.BlockSpec((BB * NP, 1), full),
        ],
        out_specs=pl.BlockSpec((BB * NP, HID), blk),
        out_shape=jax.ShapeDtypeStruct((bsz * NP, HID), jnp.float32),
    )(xp, W_embed, pet, W0, dinv_col)

    y0 = _sc_agg(z0.reshape(bsz * NP * HID)).reshape(bsz * NP, HID)

    z1 = pl.pallas_call(
        _tc2_body,
        grid=grid,
        in_specs=[
            pl.BlockSpec((BB * NP, HID), blk),
            pl.BlockSpec((BB * NP, 1), full),
            pl.BlockSpec((1, HID), full),
            pl.BlockSpec((HID, HID), full),
        ],
        out_specs=pl.BlockSpec((BB * NP, HID), blk),
        out_shape=jax.ShapeDtypeStruct((bsz * NP, HID), jnp.float32),
    )(y0, dinv_col, b0.reshape(1, HID), W1)

    y1 = _sc_agg(z1.reshape(bsz * NP * HID)).reshape(bsz * NP, HID)

    out2d = pl.pallas_call(
        _tc3_body,
        grid=grid,
        in_specs=[
            pl.BlockSpec((BB * NP, HID), blk),
            pl.BlockSpec((BB * NP, 1), full),
            pl.BlockSpec((1, HID), full),
            pl.BlockSpec((HID, ENC), full),
            pl.BlockSpec((1, ENC), full),
        ],
        out_specs=pl.BlockSpec((BB * NP, ENC), blk),
        out_shape=jax.ShapeDtypeStruct((bsz * NP, ENC), jnp.float32),
    )(y1, dinv_col, b1.reshape(1, HID), W_out, b_out.reshape(1, ENC))
    return out2d.reshape(bsz, NP, ENC)
